# baseline (speedup 1.0000x reference)
import functools
import math

import jax
import jax.numpy as jnp
from jax import lax
from jax.experimental import pallas as pl
from jax.experimental.pallas import tpu as pltpu
from jax.experimental.pallas import tpu_sc as plsc

D = 128
SEQ = 200
BATCH = 4096
VOCAB = 4
NC, NS = 2, 16
NW = NC * NS
ROWS = BATCH * SEQ
RPW = ROWS // NW
CHUNK = 128
NSTEP = RPW // CHUNK
RING = 6
PREF = 3
NITER = NSTEP // RING
TROWS = SEQ * VOCAB


def _prep_body(table_ref, pe_ref, comb_ref):
    scale = jnp.float32(math.sqrt(float(D)))
    pe = pe_ref[...]
    for v in range(VOCAB):
        comb_ref[:, v, :] = pe + table_ref[v, :][None, :] * scale


def _sc_body(comb_hbm, tok_hbm, out_hbm, idx_v,
             buf_0, buf_1, buf_2, buf_3, buf_4, buf_5, comb_sh,
             gsem_0, gsem_1, gsem_2, gsem_3, gsem_4, gsem_5,
             ssem_0, ssem_1, ssem_2, ssem_3, ssem_4, ssem_5):
    bufs = (buf_0, buf_1, buf_2, buf_3, buf_4, buf_5)
    gsems = (gsem_0, gsem_1, gsem_2, gsem_3, gsem_4, gsem_5)
    ssems = (ssem_0, ssem_1, ssem_2, ssem_3, ssem_4, ssem_5)
    cid = lax.axis_index("c")
    sid = lax.axis_index("s")
    wid = sid * NC + cid
    base = wid * RPW

    srows = 80

    @pl.when(sid < TROWS // srows)
    def _stage():
        off = pl.multiple_of(sid * srows, 8)
        pltpu.sync_copy(comb_hbm.at[pl.ds(off, srows)], buf_0.at[pl.ds(0, srows)])
        pltpu.sync_copy(buf_0.at[pl.ds(0, srows)], comb_sh.at[pl.ds(off, srows)])

    plsc.subcore_barrier()

    pltpu.sync_copy(tok_hbm.at[wid], idx_v)

    def convertstep(s):
        lanes = lax.iota(jnp.int32, 16)
        for c in range(CHUNK // 16):
            off = c * 16
            tok16 = idx_v[s, pl.ds(off, 16)]
            pos = (lanes + (s * CHUNK + off)) % SEQ
            idx_v[s, pl.ds(off, 16)] = tok16 + VOCAB * pos

    def gather(s, b, issue):
        cp = pltpu.make_async_copy(comb_sh.at[idx_v.at[s]], bufs[b], gsems[b])
        if issue:
            cp.start()
        else:
            cp.wait()

    def scatter(s, b, issue):
        cp = pltpu.make_async_copy(
            bufs[b], out_hbm.at[pl.ds(base + s * CHUNK, CHUNK)], ssems[b]
        )
        if issue:
            cp.start()
        else:
            cp.wait()

    for s in range(PREF + 1):
        convertstep(s)
    for s in range(PREF):
        gather(s, s, True)

    def body(k, carry):
        for j in range(RING):
            s = RING * k + j
            bcur = j
            bpre = (j + PREF) % RING

            if j >= PREF:
                scatter(s - PREF, (j - PREF) % RING, False)
            else:
                @pl.when(k > 0)
                def _(s=s, j=j):
                    scatter(s - PREF, (j - PREF) % RING, False)

            if RING * (NITER - 1) + j + PREF < NSTEP:
                gather(s + PREF, bpre, True)
            else:
                @pl.when(k < NITER - 1)
                def _(s=s, bpre=bpre):
                    gather(s + PREF, bpre, True)

            if RING * (NITER - 1) + j + PREF + 1 < NSTEP:
                convertstep(s + PREF + 1)
            else:
                @pl.when(k < NITER - 1)
                def _(s=s):
                    convertstep(s + PREF + 1)

            gather(s, bcur, False)
            scatter(s, bcur, True)
        return carry

    lax.fori_loop(0, NITER, body, 0)

    for s in range(RING * NITER, NSTEP):
        scatter(s - PREF, (s - PREF) % RING, False)
        gather(s, s % RING, False)
        scatter(s, s % RING, True)
    for s in range(NSTEP - PREF, NSTEP):
        scatter(s, s % RING, False)


def kernel(token_indices, table, pe):
    comb = pl.pallas_call(
        _prep_body,
        out_shape=jax.ShapeDtypeStruct((SEQ, VOCAB, D), jnp.float32),
    )(table, pe[:SEQ])

    comb = comb.reshape(TROWS, D)
    tok3 = token_indices.reshape(NW, NSTEP, CHUNK)

    sc = pl.kernel(
        _sc_body,
        out_type=jax.ShapeDtypeStruct((ROWS, D), jnp.float32),
        mesh=plsc.VectorSubcoreMesh(
            core_axis_name="c", subcore_axis_name="s", num_cores=NC, num_subcores=NS
        ),
        scratch_types=[
            pltpu.VMEM((NSTEP, CHUNK), jnp.int32),
            pltpu.VMEM((CHUNK, D), jnp.float32),
            pltpu.VMEM((CHUNK, D), jnp.float32),
            pltpu.VMEM((CHUNK, D), jnp.float32),
            pltpu.VMEM((CHUNK, D), jnp.float32),
            pltpu.VMEM((CHUNK, D), jnp.float32),
            pltpu.VMEM((CHUNK, D), jnp.float32),
            pltpu.VMEM_SHARED((TROWS, D), jnp.float32),
            pltpu.SemaphoreType.DMA,
            pltpu.SemaphoreType.DMA,
            pltpu.SemaphoreType.DMA,
            pltpu.SemaphoreType.DMA,
            pltpu.SemaphoreType.DMA,
            pltpu.SemaphoreType.DMA,
            pltpu.SemaphoreType.DMA,
            pltpu.SemaphoreType.DMA,
            pltpu.SemaphoreType.DMA,
            pltpu.SemaphoreType.DMA,
            pltpu.SemaphoreType.DMA,
            pltpu.SemaphoreType.DMA,
        ],
    )
    out = sc(comb, tok3)
    return out.reshape(BATCH, SEQ, D)


# speedup vs baseline: 1.0076x; 1.0076x over previous
import functools
import math

import jax
import jax.numpy as jnp
from jax import lax
from jax.experimental import pallas as pl
from jax.experimental.pallas import tpu as pltpu
from jax.experimental.pallas import tpu_sc as plsc

D = 128
SEQ = 200
BATCH = 4096
VOCAB = 4
NC, NS = 2, 16
NW = NC * NS
ROWS = BATCH * SEQ
RPW = ROWS // NW
CHUNK = 128
NSTEP = RPW // CHUNK
RING = 6
PREF = 3
NITER = NSTEP // RING
TROWS = SEQ * VOCAB


def _prep_body(table_ref, pe_ref, comb_ref):
    scale = jnp.float32(math.sqrt(float(D)))
    pe = pe_ref[...]
    for v in range(VOCAB):
        comb_ref[:, v, :] = pe + table_ref[v, :][None, :] * scale


def _sc_body(comb_hbm, tok_hbm, out_hbm, idx_v,
             buf_0, buf_1, buf_2, buf_3, buf_4, buf_5, comb_sh,
             gsem_0, gsem_1, gsem_2, gsem_3, gsem_4, gsem_5,
             ssem_0, ssem_1, ssem_2, ssem_3, ssem_4, ssem_5, tsem):
    bufs = (buf_0, buf_1, buf_2, buf_3, buf_4, buf_5)
    gsems = (gsem_0, gsem_1, gsem_2, gsem_3, gsem_4, gsem_5)
    ssems = (ssem_0, ssem_1, ssem_2, ssem_3, ssem_4, ssem_5)
    cid = lax.axis_index("c")
    sid = lax.axis_index("s")
    wid = sid * NC + cid
    base = wid * RPW

    tok_cp = pltpu.make_async_copy(tok_hbm.at[wid], idx_v, tsem)
    tok_cp.start()

    srows = 80

    @pl.when(sid < TROWS // srows)
    def _stage():
        off = pl.multiple_of(sid * srows, 8)
        pltpu.sync_copy(comb_hbm.at[pl.ds(off, srows)], buf_0.at[pl.ds(0, srows)])
        pltpu.sync_copy(buf_0.at[pl.ds(0, srows)], comb_sh.at[pl.ds(off, srows)])

    plsc.subcore_barrier()
    tok_cp.wait()

    def convertstep(s):
        lanes = lax.iota(jnp.int32, 16)
        for c in range(CHUNK // 16):
            off = c * 16
            tok16 = idx_v[s, pl.ds(off, 16)]
            pos = (lanes + (s * CHUNK + off)) % SEQ
            idx_v[s, pl.ds(off, 16)] = tok16 + VOCAB * pos

    def gather(s, b, issue):
        cp = pltpu.make_async_copy(comb_sh.at[idx_v.at[s]], bufs[b], gsems[b])
        if issue:
            cp.start()
        else:
            cp.wait()

    def scatter(s, b, issue):
        cp = pltpu.make_async_copy(
            bufs[b], out_hbm.at[pl.ds(base + s * CHUNK, CHUNK)], ssems[b]
        )
        if issue:
            cp.start()
        else:
            cp.wait()

    for s in range(PREF + 1):
        convertstep(s)
    for s in range(PREF):
        gather(s, s, True)

    def body(k, carry):
        for j in range(RING):
            s = RING * k + j
            bcur = j
            bpre = (j + PREF) % RING

            if j >= PREF:
                scatter(s - PREF, (j - PREF) % RING, False)
            else:
                @pl.when(k > 0)
                def _(s=s, j=j):
                    scatter(s - PREF, (j - PREF) % RING, False)

            if RING * (NITER - 1) + j + PREF < NSTEP:
                gather(s + PREF, bpre, True)
            else:
                @pl.when(k < NITER - 1)
                def _(s=s, bpre=bpre):
                    gather(s + PREF, bpre, True)

            if RING * (NITER - 1) + j + PREF + 1 < NSTEP:
                convertstep(s + PREF + 1)
            else:
                @pl.when(k < NITER - 1)
                def _(s=s):
                    convertstep(s + PREF + 1)

            gather(s, bcur, False)
            scatter(s, bcur, True)
        return carry

    lax.fori_loop(0, NITER, body, 0)

    for s in range(RING * NITER, NSTEP):
        scatter(s - PREF, (s - PREF) % RING, False)
        gather(s, s % RING, False)
        scatter(s, s % RING, True)
    for s in range(NSTEP - PREF, NSTEP):
        scatter(s, s % RING, False)


def kernel(token_indices, table, pe):
    comb = pl.pallas_call(
        _prep_body,
        out_shape=jax.ShapeDtypeStruct((SEQ, VOCAB, D), jnp.float32),
    )(table, pe[:SEQ])

    comb = comb.reshape(TROWS, D)
    tok3 = token_indices.reshape(NW, NSTEP, CHUNK)

    sc = pl.kernel(
        _sc_body,
        out_type=jax.ShapeDtypeStruct((ROWS, D), jnp.float32),
        mesh=plsc.VectorSubcoreMesh(
            core_axis_name="c", subcore_axis_name="s", num_cores=NC, num_subcores=NS
        ),
        scratch_types=[
            pltpu.VMEM((NSTEP, CHUNK), jnp.int32),
            pltpu.VMEM((CHUNK, D), jnp.float32),
            pltpu.VMEM((CHUNK, D), jnp.float32),
            pltpu.VMEM((CHUNK, D), jnp.float32),
            pltpu.VMEM((CHUNK, D), jnp.float32),
            pltpu.VMEM((CHUNK, D), jnp.float32),
            pltpu.VMEM((CHUNK, D), jnp.float32),
            pltpu.VMEM_SHARED((TROWS, D), jnp.float32),
            pltpu.SemaphoreType.DMA,
            pltpu.SemaphoreType.DMA,
            pltpu.SemaphoreType.DMA,
            pltpu.SemaphoreType.DMA,
            pltpu.SemaphoreType.DMA,
            pltpu.SemaphoreType.DMA,
            pltpu.SemaphoreType.DMA,
            pltpu.SemaphoreType.DMA,
            pltpu.SemaphoreType.DMA,
            pltpu.SemaphoreType.DMA,
            pltpu.SemaphoreType.DMA,
            pltpu.SemaphoreType.DMA,
            pltpu.SemaphoreType.DMA,
        ],
    )
    out = sc(comb, tok3)
    return out.reshape(BATCH, SEQ, D)


# speedup vs baseline: 1.0093x; 1.0017x over previous
import math

import jax
import jax.numpy as jnp
from jax import lax
from jax.experimental import pallas as pl
from jax.experimental.pallas import tpu as pltpu
from jax.experimental.pallas import tpu_sc as plsc

D = 128
SEQ = 200
BATCH = 4096
VOCAB = 4
NC, NS = 2, 16
NW = NC * NS
ROWS = BATCH * SEQ
RPW = ROWS // NW
CHUNK = 128
NSTEP = RPW // CHUNK
RING = 6
PREF = 3
NITER = NSTEP // RING
TROWS = SEQ * VOCAB


def _prep_body(table_ref, pe_ref, comb_ref):
    scale = jnp.float32(math.sqrt(float(D)))
    pe = pe_ref[...]
    for v in range(VOCAB):
        comb_ref[:, v, :] = pe + table_ref[v, :][None, :] * scale


def _sc_body(comb_hbm, tok_hbm, out_hbm, idx_v,
             buf_0, buf_1, buf_2, buf_3, buf_4, buf_5, comb_sh,
             gsem_0, gsem_1, gsem_2, gsem_3, gsem_4, gsem_5,
             ssem_0, ssem_1, ssem_2, ssem_3, ssem_4, ssem_5, tsem):
    bufs = (buf_0, buf_1, buf_2, buf_3, buf_4, buf_5)
    gsems = (gsem_0, gsem_1, gsem_2, gsem_3, gsem_4, gsem_5)
    ssems = (ssem_0, ssem_1, ssem_2, ssem_3, ssem_4, ssem_5)
    cid = lax.axis_index("c")
    sid = lax.axis_index("s")
    wid = sid * NC + cid
    base = wid * RPW

    tok_cp = pltpu.make_async_copy(tok_hbm.at[wid], idx_v, tsem)
    tok_cp.start()

    srows = 80

    @pl.when(sid < TROWS // srows)
    def _stage():
        off = pl.multiple_of(sid * srows, 8)
        pltpu.sync_copy(comb_hbm.at[pl.ds(off, srows)], buf_0.at[pl.ds(0, srows)])
        pltpu.sync_copy(buf_0.at[pl.ds(0, srows)], comb_sh.at[pl.ds(off, srows)])

    plsc.subcore_barrier()
    tok_cp.wait()

    def convertstep(s):
        lanes = lax.iota(jnp.int32, 16)
        for c in range(CHUNK // 16):
            off = c * 16
            tok16 = idx_v[s, pl.ds(off, 16)]
            pos = (lanes + (s * CHUNK + off)) % SEQ
            idx_v[s, pl.ds(off, 16)] = tok16 + VOCAB * pos

    def gather(s, b, issue):
        cp = pltpu.make_async_copy(comb_sh.at[idx_v.at[s]], bufs[b], gsems[b])
        if issue:
            cp.start()
        else:
            cp.wait()

    def scatter(s, b, issue):
        cp = pltpu.make_async_copy(
            bufs[b], out_hbm.at[pl.ds(base + s * CHUNK, CHUNK)], ssems[b]
        )
        if issue:
            cp.start()
        else:
            cp.wait()

    for s in range(PREF + 1):
        convertstep(s)
    for s in range(PREF):
        gather(s, s, True)

    def body(k, carry):
        for j in range(RING):
            s = RING * k + j
            bcur = j
            bpre = (j + PREF) % RING

            if j >= PREF:
                scatter(s - PREF, (j - PREF) % RING, False)
            else:
                @pl.when(k > 0)
                def _(s=s, j=j):
                    scatter(s - PREF, (j - PREF) % RING, False)

            if RING * (NITER - 1) + j + PREF < NSTEP:
                gather(s + PREF, bpre, True)
            else:
                @pl.when(k < NITER - 1)
                def _(s=s, bpre=bpre):
                    gather(s + PREF, bpre, True)

            if RING * (NITER - 1) + j + PREF + 1 < NSTEP:
                convertstep(s + PREF + 1)
            else:
                @pl.when(k < NITER - 1)
                def _(s=s):
                    convertstep(s + PREF + 1)

            gather(s, bcur, False)
            scatter(s, bcur, True)
        return carry

    lax.fori_loop(0, NITER, body, 0)

    for s in range(RING * NITER, NSTEP):
        scatter(s - PREF, (s - PREF) % RING, False)
        gather(s, s % RING, False)
        scatter(s, s % RING, True)
    for s in range(NSTEP - PREF, NSTEP):
        scatter(s, s % RING, False)


def kernel(token_indices, table, pe):
    comb = pl.pallas_call(
        _prep_body,
        out_shape=jax.ShapeDtypeStruct((SEQ, VOCAB, D), jnp.float32),
    )(table, pe[:SEQ])

    comb = comb.reshape(TROWS, D)
    tok3 = token_indices.reshape(NW, NSTEP, CHUNK)

    sc = pl.kernel(
        _sc_body,
        out_type=jax.ShapeDtypeStruct((ROWS, D), jnp.float32),
        mesh=plsc.VectorSubcoreMesh(
            core_axis_name="c", subcore_axis_name="s", num_cores=NC, num_subcores=NS
        ),
        scratch_types=[
            pltpu.VMEM((NSTEP, CHUNK), jnp.int32),
            pltpu.VMEM((CHUNK, D), jnp.float32),
            pltpu.VMEM((CHUNK, D), jnp.float32),
            pltpu.VMEM((CHUNK, D), jnp.float32),
            pltpu.VMEM((CHUNK, D), jnp.float32),
            pltpu.VMEM((CHUNK, D), jnp.float32),
            pltpu.VMEM((CHUNK, D), jnp.float32),
            pltpu.VMEM_SHARED((TROWS, D), jnp.float32),
            pltpu.SemaphoreType.DMA,
            pltpu.SemaphoreType.DMA,
            pltpu.SemaphoreType.DMA,
            pltpu.SemaphoreType.DMA,
            pltpu.SemaphoreType.DMA,
            pltpu.SemaphoreType.DMA,
            pltpu.SemaphoreType.DMA,
            pltpu.SemaphoreType.DMA,
            pltpu.SemaphoreType.DMA,
            pltpu.SemaphoreType.DMA,
            pltpu.SemaphoreType.DMA,
            pltpu.SemaphoreType.DMA,
            pltpu.SemaphoreType.DMA,
        ],
    )
    out = sc(comb, tok3)
    return out.reshape(BATCH, SEQ, D)
